# Initial kernel scaffold; baseline (speedup 1.0000x reference)
import jax
import jax.numpy as jnp
from jax.experimental import pallas as pl


def kernel(user_emb, item_emb, user_intent, item_intent, norm_vals, h_idx, t_idx, users, items):
    raise NotImplementedError("write your pallas kernel here")


# speedup vs baseline: 4.3640x; 4.3640x over previous
import functools

import jax
import jax.numpy as jnp
from jax import lax
from jax.experimental import pallas as pl
from jax.experimental.pallas import tpu as pltpu
from jax.experimental.pallas import tpu_sc as plsc

NU = 5000
N = 10000
D = 128
E = 320000
NC = 2
NS = 16
NW = NC * NS
C = 128
NCHUNKS = E // C
U = 40
NUNITS = N // U
CE = 80
NCHE = E // CE
_MESH = plsc.VectorSubcoreMesh(core_axis_name="c", subcore_axis_name="s")
_SC_PARAMS = pltpu.CompilerParams(needs_layout_passes=False)


def _zero_rows(rows, nrows):
    z = jnp.zeros((16,), jnp.float32)
    w = rows.shape[1]

    def zrow(i, c):
        for j in range(w // 16):
            rows[i, pl.ds(j * 16, 16)] = z
        return c

    lax.fori_loop(0, nrows, zrow, 0)


def _unit_count(sid):
    nfull = NUNITS // NS
    return jnp.where(sid < NUNITS - nfull * NS, nfull + 1, nfull)


def _unit_loop(sid, fn):
    def step(k, c):
        fn((sid + k * NS) * U)
        return c

    lax.fori_loop(0, _unit_count(sid), step, 0)


def _spmm_body(x_hbm, h_hbm, t_hbm, v_hbm, out_hbm, acc, hv, tv, vv, rows, sem):
    cid = lax.axis_index("c")
    sid = lax.axis_index("s")
    wid = sid * NC + cid

    _zero_rows(rows, U)
    _unit_loop(sid, lambda off: pltpu.sync_copy(
        rows.at[pl.ds(0, U)], acc.at[pl.ds(off, U)]))
    plsc.subcore_barrier()

    nfull = NCHUNKS // NW
    nch = jnp.where(wid < NCHUNKS - nfull * NW, nfull + 1, nfull)

    def chunk(k, c):
        base = (wid + k * NW) * C
        pltpu.sync_copy(h_hbm.at[pl.ds(base, C)], hv)
        pltpu.sync_copy(t_hbm.at[pl.ds(base, C)], tv)
        pltpu.sync_copy(v_hbm.at[pl.ds(base, C)], vv)
        pltpu.async_copy(x_hbm.at[tv], rows, sem).wait()

        def edge16(g, cc):
            vals = vv[pl.ds(g * 16, 16)]
            for l in range(16):
                e = g * 16 + l
                val = vals[l]
                for j in range(D // 16):
                    sl = pl.ds(j * 16, 16)
                    rows[e, sl] = rows[e, sl] * val
            return cc

        lax.fori_loop(0, C // 16, edge16, 0)
        pltpu.sync_copy(rows, acc.at[hv], add=True)
        return c

    lax.fori_loop(0, nch, chunk, 0)
    plsc.subcore_barrier()
    _unit_loop(sid, lambda off: pltpu.sync_copy(
        acc.at[pl.ds(off, U)], out_hbm.at[cid, pl.ds(off, U)]))


@jax.jit
def _spmm_sc(x, h_idx, t_idx, vals):
    k = pl.kernel(
        _spmm_body,
        out_type=jax.ShapeDtypeStruct((NC, N, D), jnp.float32),
        mesh=_MESH,
        compiler_params=_SC_PARAMS,
        scratch_types=[
            pltpu.VMEM_SHARED((N, D), jnp.float32),
            pltpu.VMEM((C,), jnp.int32),
            pltpu.VMEM((C,), jnp.int32),
            pltpu.VMEM((C,), jnp.float32),
            pltpu.VMEM((C, D), jnp.float32),
            pltpu.SemaphoreType.DMA,
        ],
    )
    return k(x, h_idx, t_idx, vals)


def _alpha_body(cat_hbm, h_hbm, t_hbm, alpha_hbm, rs_hbm,
                rs, hv, tv, hcv, av, rh, rt, pt, sem):
    cid = lax.axis_index("c")
    sid = lax.axis_index("s")
    coff = cid * N

    _zero_rows(rh, U)
    _unit_loop(sid, lambda off: pltpu.sync_copy(
        rh.at[pl.ds(0, U)], rs.at[pl.ds(off, U)]))
    plsc.subcore_barrier()

    def chunk(k, c):
        base = (sid + k * NS) * CE
        pltpu.sync_copy(h_hbm.at[pl.ds(base, CE)], hv)
        pltpu.sync_copy(t_hbm.at[pl.ds(base, CE)], tv)
        for j in range(CE // 16):
            sl = pl.ds(j * 16, 16)
            hcv[sl] = hv[sl] + coff
            tcv = tv[sl] + coff
            tv[sl] = tcv
        d1 = pltpu.async_copy(cat_hbm.at[hcv], rh, sem)
        d2 = pltpu.async_copy(cat_hbm.at[tv], rt, sem)
        d1.wait()
        d2.wait()

        def grp(g, cc):
            for l in range(16):
                e = g * 16 + l
                p = rh[e, pl.ds(0, 16)] * rt[e, pl.ds(0, 16)]
                for j in range(1, D // 16):
                    sl = pl.ds(j * 16, 16)
                    p = p + rh[e, sl] * rt[e, sl]
                pt[l, :] = p
            lanes = lax.broadcasted_iota(jnp.int32, (16,), 0)
            s = plsc.load_gather(pt, [lanes, jnp.zeros((16,), jnp.int32)])
            for j in range(1, 16):
                s = s + plsc.load_gather(pt, [lanes, jnp.full((16,), j, jnp.int32)])
            avec = (s + 1.0) * 0.5
            av[pl.ds(g * 16, 16)] = avec
            for l in range(16):
                rh[g * 16 + l, pl.ds(0, 16)] = avec[l] + jnp.zeros((16,), jnp.float32)
            return cc

        lax.fori_loop(0, CE // 16, grp, 0)
        pltpu.sync_copy(av, alpha_hbm.at[pl.ds(cid * E + base, CE)])
        pltpu.sync_copy(rh, rs.at[hv], add=True)
        return c

    lax.fori_loop(0, NCHE // NS, chunk, 0)
    plsc.subcore_barrier()
    _unit_loop(sid, lambda off: pltpu.sync_copy(
        rs.at[pl.ds(off, U)], rs_hbm.at[cid, pl.ds(off, U)]))


@jax.jit
def _alpha_sc(cat, h_idx, t_idx):
    k = pl.kernel(
        _alpha_body,
        out_type=(jax.ShapeDtypeStruct((NC * E,), jnp.float32),
                  jax.ShapeDtypeStruct((NC, N, D), jnp.float32)),
        mesh=_MESH,
        compiler_params=_SC_PARAMS,
        scratch_types=[
            pltpu.VMEM_SHARED((N, D), jnp.float32),
            pltpu.VMEM((CE,), jnp.int32),
            pltpu.VMEM((CE,), jnp.int32),
            pltpu.VMEM((CE,), jnp.int32),
            pltpu.VMEM((CE,), jnp.float32),
            pltpu.VMEM((CE, D), jnp.float32),
            pltpu.VMEM((CE, D), jnp.float32),
            pltpu.VMEM((16, 16), jnp.float32),
            pltpu.SemaphoreType.DMA,
        ],
    )
    return k(cat, h_idx, t_idx)


def _apply_body(x_hbm, h_hbm, t_hbm, alpha_hbm, out_hbm,
                acc, hv, tv, av, rows, sem):
    cid = lax.axis_index("c")
    sid = lax.axis_index("s")

    _zero_rows(rows, U)
    _unit_loop(sid, lambda off: pltpu.sync_copy(
        rows.at[pl.ds(0, U)], acc.at[pl.ds(off, U)]))
    plsc.subcore_barrier()

    def chunk(k, c):
        base = (sid + k * NS) * CE
        pltpu.sync_copy(h_hbm.at[pl.ds(base, CE)], hv)
        pltpu.sync_copy(t_hbm.at[pl.ds(base, CE)], tv)
        pltpu.sync_copy(alpha_hbm.at[pl.ds(cid * E + base, CE)], av)
        pltpu.async_copy(x_hbm.at[tv], rows, sem).wait()

        def edge16(g, cc):
            vals = av[pl.ds(g * 16, 16)]
            for l in range(16):
                e = g * 16 + l
                val = vals[l]
                for j in range(D // 16):
                    sl = pl.ds(j * 16, 16)
                    rows[e, sl] = rows[e, sl] * val
            return cc

        lax.fori_loop(0, CE // 16, edge16, 0)
        pltpu.sync_copy(rows, acc.at[hv], add=True)
        return c

    lax.fori_loop(0, NCHE // NS, chunk, 0)
    plsc.subcore_barrier()
    _unit_loop(sid, lambda off: pltpu.sync_copy(
        acc.at[pl.ds(off, U)], out_hbm.at[cid, pl.ds(off, U)]))


@jax.jit
def _apply_sc(x, h_idx, t_idx, alphas):
    k = pl.kernel(
        _apply_body,
        out_type=jax.ShapeDtypeStruct((NC, N, D), jnp.float32),
        mesh=_MESH,
        compiler_params=_SC_PARAMS,
        scratch_types=[
            pltpu.VMEM_SHARED((N, D), jnp.float32),
            pltpu.VMEM((CE,), jnp.int32),
            pltpu.VMEM((CE,), jnp.int32),
            pltpu.VMEM((CE,), jnp.float32),
            pltpu.VMEM((CE, D), jnp.float32),
            pltpu.SemaphoreType.DMA,
        ],
    )
    return k(x, h_idx, t_idx, alphas)


RB = 1000


def _dense_body(x_ref, p0_ref, p1_ref, uw_ref, iw_ref,
                gnn_ref, int_ref, gn_ref, in_ref):
    pid = pl.program_id(0)
    g = p0_ref[...] + p1_ref[...]
    w = jnp.where(pid < (NU // RB), uw_ref[...], iw_ref[...])
    logits = jnp.dot(x_ref[...], w, preferred_element_type=jnp.float32)
    m = jnp.max(logits, axis=1, keepdims=True)
    ee = jnp.exp(logits - m)
    sm = ee / jnp.sum(ee, axis=1, keepdims=True)
    il = lax.dot_general(sm, w, dimension_numbers=(((1,), (1,)), ((), ())),
                         preferred_element_type=jnp.float32)
    gnn_ref[...] = g
    int_ref[...] = il
    gn_ref[...] = g / jnp.maximum(
        jnp.sqrt(jnp.sum(g * g, axis=1, keepdims=True)), 1e-12)
    in_ref[...] = il / jnp.maximum(
        jnp.sqrt(jnp.sum(il * il, axis=1, keepdims=True)), 1e-12)


@jax.jit
def _dense_tc(x, gnn_p, user_intent, item_intent):
    blk = pl.BlockSpec((RB, D), lambda i: (i, 0))
    wblk = pl.BlockSpec((D, D), lambda i: (0, 0))
    sds = jax.ShapeDtypeStruct((N, D), jnp.float32)
    return pl.pallas_call(
        _dense_body,
        grid=(N // RB,),
        in_specs=[blk, blk, blk, wblk, wblk],
        out_specs=[blk, blk, blk, blk],
        out_shape=[sds, sds, sds, sds],
    )(x, gnn_p[0], gnn_p[1], user_intent, item_intent)


def _combine_body(gnn_ref, int_ref, ag_ref, ai_ref, rsg_ref, rsi_ref, x_ref,
                  gaa_ref, iaa_ref, xn_ref):
    rsg = rsg_ref[..., 0:1]
    rsi = rsi_ref[..., 0:1]
    dg = jnp.where(rsg > 0, 1.0 / rsg, 0.0)
    di = jnp.where(rsi > 0, 1.0 / rsi, 0.0)
    gaa = dg * ag_ref[...]
    iaa = di * ai_ref[...]
    gaa_ref[...] = gaa
    iaa_ref[...] = iaa
    xn_ref[...] = gnn_ref[...] + int_ref[...] + gaa + iaa + x_ref[...]


def _combine_total_body(gnn_ref, int_ref, ag_ref, ai_ref, rsg_ref, rsi_ref,
                        x_ref, x0_ref, gaa_ref, iaa_ref, xn_ref, tot_ref):
    _combine_body(gnn_ref, int_ref, ag_ref, ai_ref, rsg_ref, rsi_ref, x_ref,
                  gaa_ref, iaa_ref, xn_ref)
    tot_ref[...] = x0_ref[...] + x_ref[...] + xn_ref[...]


@functools.partial(jax.jit, static_argnames=("with_total",))
def _combine_tc(gnn, intl, accp, rsp, x, x0=None, with_total=False):
    blk = pl.BlockSpec((RB, D), lambda i: (i, 0))
    rblk = pl.BlockSpec((RB, D), lambda i: (i, 0))
    sds = jax.ShapeDtypeStruct((N, D), jnp.float32)
    nin = 7 + (1 if with_total else 0)
    nout = 3 + (1 if with_total else 0)
    body = _combine_total_body if with_total else _combine_body
    args = [gnn, intl, accp[0], accp[1], rsp[0], rsp[1], x]
    if with_total:
        args.append(x0)
    return pl.pallas_call(
        body,
        grid=(N // RB,),
        in_specs=[blk] * 4 + [rblk] * 2 + [blk] * (nin - 6),
        out_specs=[blk] * nout,
        out_shape=[sds] * nout,
    )(*args)


def kernel(user_emb, item_emb, user_intent, item_intent, norm_vals,
           h_idx, t_idx, users, items):
    x0 = jnp.concatenate([user_emb, item_emb], axis=0)
    h_idx = h_idx.astype(jnp.int32)
    t_idx = t_idx.astype(jnp.int32)

    x = x0
    gnn_l, int_l, gaa_l, iaa_l = [], [], [], []
    for layer in range(2):
        gnn_p = _spmm_sc(x, h_idx, t_idx, norm_vals)
        gnn, intl, gn, inn = _dense_tc(x, gnn_p, user_intent, item_intent)
        cat = jnp.concatenate([gn, inn], axis=0)
        alphas, rsp = _alpha_sc(cat, h_idx, t_idx)
        accp = _apply_sc(x, h_idx, t_idx, alphas)
        if layer == 0:
            gaa, iaa, xn = _combine_tc(gnn, intl, accp, rsp, x)
        else:
            gaa, iaa, xn, tot = _combine_tc(gnn, intl, accp, rsp, x, x0,
                                            with_total=True)
        gnn_l.append(gnn)
        int_l.append(intl)
        gaa_l.append(gaa)
        iaa_l.append(iaa)
        x = xn

    return (tot[:NU], tot[NU:], tuple(gnn_l), tuple(int_l),
            tuple(gaa_l), tuple(iaa_l))


# speedup vs baseline: 6.6375x; 1.5210x over previous
import functools

import jax
import jax.numpy as jnp
from jax import lax
from jax.experimental import pallas as pl
from jax.experimental.pallas import tpu as pltpu
from jax.experimental.pallas import tpu_sc as plsc

NU = 5000
N = 10000
NPAD = 10240
D = 128
E = 320000
NC = 2
NS = 16
NW = NC * NS
CE = 80
NCHA = E // CE // NS
NCHS = E // CE // NW
U = 40
NUNITS = N // U
_MESH = plsc.VectorSubcoreMesh(core_axis_name="c", subcore_axis_name="s")
_SC_PARAMS = pltpu.CompilerParams(needs_layout_passes=False)


def _zero_rows(rows, nrows):
    z = jnp.zeros((16,), jnp.float32)
    w = rows.shape[1]

    def zrow(i, c):
        for j in range(w // 16):
            rows[i, pl.ds(j * 16, 16)] = z
        return c

    lax.fori_loop(0, nrows, zrow, 0)


def _unit_count(sid):
    nfull = NUNITS // NS
    return jnp.where(sid < NUNITS - nfull * NS, nfull + 1, nfull)


def _unit_loop(sid, fn):
    def step(k, c):
        fn((sid + k * NS) * U)
        return c

    lax.fori_loop(0, _unit_count(sid), step, 0)


def _make_wspmm_body(all_edges):
    def body(x_hbm, h_hbm, t_hbm, w_hbm, out_hbm, acc,
             tv0, tv1, hn0, hn1, hv0, hv1, av0, av1,
             rows0, rows1, sc0, sc1, sg0, sg1, ss0, ss1):
        cid = lax.axis_index("c")
        sid = lax.axis_index("s")
        TV = (tv0, tv1)
        HN = (hn0, hn1)
        HV = (hv0, hv1)
        AV = (av0, av1)
        ROWS = (rows0, rows1)
        SCB = (sc0, sc1)
        SG = (sg0, sg1)
        SS = (ss0, ss1)

        _zero_rows(sc0, U)
        _unit_loop(sid, lambda off: pltpu.sync_copy(
            sc0.at[pl.ds(0, U)], acc.at[pl.ds(off, U)]))
        plsc.subcore_barrier()

        if all_edges:
            woff = cid * E
            nturn = NCHA

            def base_of(k):
                return (sid + k * NS) * CE
        else:
            wid = sid * NC + cid
            woff = 0
            nturn = NCHS

            def base_of(k):
                return (wid + k * NW) * CE

        def prep(b, k):
            base = base_of(k)
            pltpu.sync_copy(t_hbm.at[pl.ds(base, CE)], TV[b])
            pltpu.sync_copy(h_hbm.at[pl.ds(base, CE)], HN[b])
            pltpu.sync_copy(w_hbm.at[pl.ds(woff + base, CE)], AV[b])
            pltpu.async_copy(x_hbm.at[TV[b]], ROWS[b], SG[b])

        for b in range(2):
            prep(b, b)

        def turn(s, b, k):
            pltpu.make_async_copy(x_hbm.at[TV[b]], ROWS[b], SG[b]).wait()

            @pl.when(s > 0)
            def _():
                pltpu.make_async_copy(SCB[b], acc.at[HV[b]], SS[b]).wait()

            for j in range(CE // 16):
                sl = pl.ds(j * 16, 16)
                HV[b][sl] = HN[b][sl]

            def grp(g, c):
                vals = AV[b][pl.ds(g * 16, 16)]
                for l in range(16):
                    e = g * 16 + l
                    val = vals[l]
                    for j in range(D // 16):
                        sl = pl.ds(j * 16, 16)
                        SCB[b][e, sl] = ROWS[b][e, sl] * val
                return c

            lax.fori_loop(0, CE // 16, grp, 0)
            pltpu.async_copy(SCB[b], acc.at[HV[b]], SS[b], add=True)

            @pl.when(k + 2 < nturn)
            def _():
                prep(b, k + 2)

        def step(s, c):
            turn(s, 0, 2 * s)
            turn(s, 1, 2 * s + 1)
            return c

        lax.fori_loop(0, nturn // 2, step, 0)
        if nturn % 2:
            turn(jnp.int32(nturn // 2), 0, jnp.int32(nturn - 1))
        pltpu.make_async_copy(SCB[0], acc.at[HV[0]], SS[0]).wait()
        pltpu.make_async_copy(SCB[1], acc.at[HV[1]], SS[1]).wait()

        plsc.subcore_barrier()
        _unit_loop(sid, lambda off: pltpu.sync_copy(
            acc.at[pl.ds(off, U)], out_hbm.at[cid, pl.ds(off, U)]))

    return body


def _wspmm_scratch():
    return [
        pltpu.VMEM_SHARED((N, D), jnp.float32),
        pltpu.VMEM((CE,), jnp.int32),
        pltpu.VMEM((CE,), jnp.int32),
        pltpu.VMEM((CE,), jnp.int32),
        pltpu.VMEM((CE,), jnp.int32),
        pltpu.VMEM((CE,), jnp.int32),
        pltpu.VMEM((CE,), jnp.int32),
        pltpu.VMEM((CE,), jnp.float32),
        pltpu.VMEM((CE,), jnp.float32),
        pltpu.VMEM((CE, D), jnp.float32),
        pltpu.VMEM((CE, D), jnp.float32),
        pltpu.VMEM((CE, D), jnp.float32),
        pltpu.VMEM((CE, D), jnp.float32),
        pltpu.SemaphoreType.DMA,
        pltpu.SemaphoreType.DMA,
        pltpu.SemaphoreType.DMA,
        pltpu.SemaphoreType.DMA,
    ]


@jax.jit
def _spmm_sc(x, h_idx, t_idx, vals):
    k = pl.kernel(
        _make_wspmm_body(all_edges=False),
        out_type=jax.ShapeDtypeStruct((NC, N, D), jnp.float32),
        mesh=_MESH,
        compiler_params=_SC_PARAMS,
        scratch_types=_wspmm_scratch(),
    )
    return k(x, h_idx, t_idx, vals)


@jax.jit
def _apply_sc(x, h_idx, t_idx, alphas):
    k = pl.kernel(
        _make_wspmm_body(all_edges=True),
        out_type=jax.ShapeDtypeStruct((NC, N, D), jnp.float32),
        mesh=_MESH,
        compiler_params=_SC_PARAMS,
        scratch_types=_wspmm_scratch(),
    )
    return k(x, h_idx, t_idx, alphas)


def _alpha_body(cat_hbm, h_hbm, t_hbm, alpha_hbm, rs_hbm,
                stage, rsloc, hr0, hr1, th0, th1, tt0, tt1, av0, av1,
                rh0, rh1, rt0, rt1, pt, redbuf, outv,
                sg0, sg1, sw0, sw1):
    cid = lax.axis_index("c")
    sid = lax.axis_index("s")
    coff = cid * N
    HR = (hr0, hr1)
    TH = (th0, th1)
    TT = (tt0, tt1)
    AV = (av0, av1)
    RH = (rh0, rh1)
    RT = (rt0, rt1)
    SG = (sg0, sg1)
    SW = (sw0, sw1)

    def z(i, c):
        rsloc[pl.ds(i * 16, 16)] = jnp.zeros((16,), jnp.float32)
        return c

    lax.fori_loop(0, NPAD // 16, z, 0)

    def base_of(k):
        return (sid + k * NS) * CE

    def prep(b, k):
        base = base_of(k)
        pltpu.sync_copy(h_hbm.at[pl.ds(base, CE)], HR[b])
        pltpu.sync_copy(t_hbm.at[pl.ds(base, CE)], TT[b])
        for j in range(CE // 16):
            sl = pl.ds(j * 16, 16)
            TH[b][sl] = HR[b][sl] + coff
            TT[b][sl] = TT[b][sl] + coff
        pltpu.async_copy(cat_hbm.at[TH[b]], RH[b], SG[b])
        pltpu.async_copy(cat_hbm.at[TT[b]], RT[b], SG[b])

    for b in range(2):
        prep(b, b)

    lanes = lax.broadcasted_iota(jnp.int32, (16,), 0)

    def turn(s, b):
        k = 2 * s + b
        base = base_of(k)
        pltpu.make_async_copy(cat_hbm.at[TH[b]], RH[b], SG[b]).wait()
        pltpu.make_async_copy(cat_hbm.at[TT[b]], RT[b], SG[b]).wait()

        @pl.when(s > 0)
        def _():
            pltpu.make_async_copy(
                AV[b], alpha_hbm.at[pl.ds(cid * E + base, CE)], SW[b]).wait()

        def grp(g, c):
            for l in range(16):
                e = g * 16 + l
                p = RH[b][e, pl.ds(0, 16)] * RT[b][e, pl.ds(0, 16)]
                for j in range(1, D // 16):
                    sl = pl.ds(j * 16, 16)
                    p = p + RH[b][e, sl] * RT[b][e, sl]
                pt[l, :] = p
            sv = plsc.load_gather(pt, [lanes, jnp.zeros((16,), jnp.int32)])
            for j in range(1, 16):
                sv = sv + plsc.load_gather(pt, [lanes, jnp.full((16,), j, jnp.int32)])
            avec = (sv + 1.0) * 0.5
            AV[b][pl.ds(g * 16, 16)] = avec
            hvec = HR[b][pl.ds(g * 16, 16)]
            plsc.addupdate_scatter(rsloc, [hvec], avec)
            return c

        lax.fori_loop(0, CE // 16, grp, 0)
        pltpu.async_copy(AV[b], alpha_hbm.at[pl.ds(cid * E + base, CE)], SW[b])

        @pl.when(k + 2 < NCHA)
        def _():
            prep(b, k + 2)

    def step(s, c):
        turn(s, 0)
        turn(s, 1)
        return c

    lax.fori_loop(0, NCHA // 2, step, 0)
    for b in range(2):
        base = base_of(jnp.int32(NCHA - 2 + b))
        pltpu.make_async_copy(
            AV[b], alpha_hbm.at[pl.ds(cid * E + base, CE)], SW[b]).wait()

    pltpu.sync_copy(rsloc, stage.at[sid])
    plsc.subcore_barrier()
    pltpu.sync_copy(stage.at[:, pl.ds(sid * 640, 640)], redbuf)

    def red(kk, c):
        sl = pl.ds(kk * 16, 16)
        sv = redbuf[0, sl]
        for r in range(1, 16):
            sv = sv + redbuf[r, sl]
        outv[sl] = sv
        return c

    lax.fori_loop(0, 40, red, 0)
    pltpu.sync_copy(outv, rs_hbm.at[pl.ds(cid * NPAD + sid * 640, 640)])


@jax.jit
def _alpha_sc(cat, h_idx, t_idx):
    k = pl.kernel(
        _alpha_body,
        out_type=(jax.ShapeDtypeStruct((NC * E,), jnp.float32),
                  jax.ShapeDtypeStruct((NC * NPAD,), jnp.float32)),
        mesh=_MESH,
        compiler_params=_SC_PARAMS,
        scratch_types=[
            pltpu.VMEM_SHARED((NS, NPAD), jnp.float32),
            pltpu.VMEM((NPAD,), jnp.float32),
            pltpu.VMEM((CE,), jnp.int32),
            pltpu.VMEM((CE,), jnp.int32),
            pltpu.VMEM((CE,), jnp.int32),
            pltpu.VMEM((CE,), jnp.int32),
            pltpu.VMEM((CE,), jnp.int32),
            pltpu.VMEM((CE,), jnp.int32),
            pltpu.VMEM((CE,), jnp.float32),
            pltpu.VMEM((CE,), jnp.float32),
            pltpu.VMEM((CE, D), jnp.float32),
            pltpu.VMEM((CE, D), jnp.float32),
            pltpu.VMEM((CE, D), jnp.float32),
            pltpu.VMEM((CE, D), jnp.float32),
            pltpu.VMEM((16, 16), jnp.float32),
            pltpu.VMEM((16, 640), jnp.float32),
            pltpu.VMEM((640,), jnp.float32),
            pltpu.SemaphoreType.DMA,
            pltpu.SemaphoreType.DMA,
            pltpu.SemaphoreType.DMA,
            pltpu.SemaphoreType.DMA,
        ],
    )
    return k(cat, h_idx, t_idx)


RB = 1000


def _dense_body(x_ref, p0_ref, p1_ref, uw_ref, iw_ref,
                gnn_ref, int_ref, gn_ref, in_ref):
    pid = pl.program_id(0)
    g = p0_ref[...] + p1_ref[...]
    w = jnp.where(pid < (NU // RB), uw_ref[...], iw_ref[...])
    logits = jnp.dot(x_ref[...], w, preferred_element_type=jnp.float32)
    m = jnp.max(logits, axis=1, keepdims=True)
    ee = jnp.exp(logits - m)
    sm = ee / jnp.sum(ee, axis=1, keepdims=True)
    il = lax.dot_general(sm, w, dimension_numbers=(((1,), (1,)), ((), ())),
                         preferred_element_type=jnp.float32)
    gnn_ref[...] = g
    int_ref[...] = il
    gn_ref[...] = g / jnp.maximum(
        jnp.sqrt(jnp.sum(g * g, axis=1, keepdims=True)), 1e-12)
    in_ref[...] = il / jnp.maximum(
        jnp.sqrt(jnp.sum(il * il, axis=1, keepdims=True)), 1e-12)


@jax.jit
def _dense_tc(x, gnn_p, user_intent, item_intent):
    blk = pl.BlockSpec((RB, D), lambda i: (i, 0))
    wblk = pl.BlockSpec((D, D), lambda i: (0, 0))
    sds = jax.ShapeDtypeStruct((N, D), jnp.float32)
    return pl.pallas_call(
        _dense_body,
        grid=(N // RB,),
        in_specs=[blk, blk, blk, wblk, wblk],
        out_specs=[blk, blk, blk, blk],
        out_shape=[sds, sds, sds, sds],
    )(x, gnn_p[0], gnn_p[1], user_intent, item_intent)


def _combine_body(gnn_ref, int_ref, ag_ref, ai_ref, rsg_ref, rsi_ref, x_ref,
                  gaa_ref, iaa_ref, xn_ref):
    rsg = rsg_ref[...]
    rsi = rsi_ref[...]
    dg = jnp.where(rsg > 0, 1.0 / rsg, 0.0)
    di = jnp.where(rsi > 0, 1.0 / rsi, 0.0)
    gaa = dg * ag_ref[...]
    iaa = di * ai_ref[...]
    gaa_ref[...] = gaa
    iaa_ref[...] = iaa
    xn_ref[...] = gnn_ref[...] + int_ref[...] + gaa + iaa + x_ref[...]


def _combine_total_body(gnn_ref, int_ref, ag_ref, ai_ref, rsg_ref, rsi_ref,
                        x_ref, x0_ref, gaa_ref, iaa_ref, xn_ref, tot_ref):
    _combine_body(gnn_ref, int_ref, ag_ref, ai_ref, rsg_ref, rsi_ref, x_ref,
                  gaa_ref, iaa_ref, xn_ref)
    tot_ref[...] = x0_ref[...] + x_ref[...] + xn_ref[...]


@functools.partial(jax.jit, static_argnames=("with_total",))
def _combine_tc(gnn, intl, accp, rsg, rsi, x, x0=None, with_total=False):
    blk = pl.BlockSpec((RB, D), lambda i: (i, 0))
    rblk = pl.BlockSpec((RB, 1), lambda i: (i, 0))
    sds = jax.ShapeDtypeStruct((N, D), jnp.float32)
    nout = 3 + (1 if with_total else 0)
    body = _combine_total_body if with_total else _combine_body
    args = [gnn, intl, accp[0], accp[1], rsg, rsi, x]
    if with_total:
        args.append(x0)
    nblk = len(args) - 6
    return pl.pallas_call(
        body,
        grid=(N // RB,),
        in_specs=[blk] * 4 + [rblk] * 2 + [blk] * nblk,
        out_specs=[blk] * nout,
        out_shape=[sds] * nout,
    )(*args)


def kernel(user_emb, item_emb, user_intent, item_intent, norm_vals,
           h_idx, t_idx, users, items):
    x0 = jnp.concatenate([user_emb, item_emb], axis=0)
    h_idx = h_idx.astype(jnp.int32)
    t_idx = t_idx.astype(jnp.int32)

    x = x0
    gnn_l, int_l, gaa_l, iaa_l = [], [], [], []
    for layer in range(2):
        gnn_p = _spmm_sc(x, h_idx, t_idx, norm_vals)
        gnn, intl, gn, inn = _dense_tc(x, gnn_p, user_intent, item_intent)
        cat = jnp.concatenate([gn, inn], axis=0)
        alphas, rs = _alpha_sc(cat, h_idx, t_idx)
        accp = _apply_sc(x, h_idx, t_idx, alphas)
        rsg = rs[:N][:, None]
        rsi = rs[NPAD:NPAD + N][:, None]
        if layer == 0:
            gaa, iaa, xn = _combine_tc(gnn, intl, accp, rsg, rsi, x)
        else:
            gaa, iaa, xn, tot = _combine_tc(gnn, intl, accp, rsg, rsi, x, x0,
                                            with_total=True)
        gnn_l.append(gnn)
        int_l.append(intl)
        gaa_l.append(gaa)
        iaa_l.append(iaa)
        x = xn

    return (tot[:NU], tot[NU:], tuple(gnn_l), tuple(int_l),
            tuple(gaa_l), tuple(iaa_l))


# speedup vs baseline: 8.9677x; 1.3511x over previous
import functools

import jax
import jax.numpy as jnp
from jax import lax
from jax.experimental import pallas as pl
from jax.experimental.pallas import tpu as pltpu
from jax.experimental.pallas import tpu_sc as plsc

NU = 5000
N = 10000
NPAD = 10240
D = 128
E = 320000
NC = 2
NS = 16
NW = NC * NS
CE = 80
NCHA = E // CE // NS
NCHS = E // CE // NW
U = 40
NUNITS = N // U
_MESH = plsc.VectorSubcoreMesh(core_axis_name="c", subcore_axis_name="s")
_SC_PARAMS = pltpu.CompilerParams(needs_layout_passes=False)


def _zero_rows(rows, nrows):
    z = jnp.zeros((16,), jnp.float32)
    w = rows.shape[1]

    def zrow(i, c):
        for j in range(w // 16):
            rows[i, pl.ds(j * 16, 16)] = z
        return c

    lax.fori_loop(0, nrows, zrow, 0)


def _unit_count(sid):
    nfull = NUNITS // NS
    return jnp.where(sid < NUNITS - nfull * NS, nfull + 1, nfull)


def _unit_loop(sid, fn):
    def step(k, c):
        fn((sid + k * NS) * U)
        return c

    lax.fori_loop(0, _unit_count(sid), step, 0)


def _make_wspmm_body(all_edges):
    def body(x_hbm, h_hbm, t_hbm, w_hbm, out_hbm, acc,
             tv0, tv1, hn0, hn1, hv0, hv1, av0, av1,
             rows0, rows1, sc0, sc1, sg0, sg1, ss0, ss1, si0, si1):
        cid = lax.axis_index("c")
        sid = lax.axis_index("s")
        TV = (tv0, tv1)
        HN = (hn0, hn1)
        HV = (hv0, hv1)
        AV = (av0, av1)
        ROWS = (rows0, rows1)
        SCB = (sc0, sc1)
        SG = (sg0, sg1)
        SS = (ss0, ss1)
        SI = (si0, si1)

        _zero_rows(sc0, U)
        _unit_loop(sid, lambda off: pltpu.sync_copy(
            sc0.at[pl.ds(0, U)], acc.at[pl.ds(off, U)]))
        plsc.subcore_barrier()

        if all_edges:
            woff = cid * E
            nturn = NCHA

            def base_of(k):
                return (sid + k * NS) * CE
        else:
            wid = sid * NC + cid
            woff = 0
            nturn = NCHS

            def base_of(k):
                return (wid + k * NW) * CE

        def prep_idx(b, k):
            base = base_of(k)
            pltpu.async_copy(t_hbm.at[pl.ds(base, CE)], TV[b], SI[b])
            pltpu.async_copy(h_hbm.at[pl.ds(base, CE)], HN[b], SI[b])
            pltpu.async_copy(w_hbm.at[pl.ds(woff + base, CE)], AV[b], SI[b])

        def wait_idx(b, k):
            base = base_of(k)
            pltpu.make_async_copy(t_hbm.at[pl.ds(base, CE)], TV[b], SI[b]).wait()
            pltpu.make_async_copy(h_hbm.at[pl.ds(base, CE)], HN[b], SI[b]).wait()
            pltpu.make_async_copy(w_hbm.at[pl.ds(woff + base, CE)], AV[b], SI[b]).wait()

        prep_idx(0, 0)
        prep_idx(1, 1)
        wait_idx(0, jnp.int32(0))
        pltpu.async_copy(x_hbm.at[TV[0]], ROWS[0], SG[0])

        def turn(s, b, k, last):
            bn = 1 - b
            if not last:
                @pl.when(k + 1 < nturn)
                def _():
                    wait_idx(bn, k + 1)
                    pltpu.async_copy(x_hbm.at[TV[bn]], ROWS[bn], SG[bn])
            pltpu.make_async_copy(x_hbm.at[TV[b]], ROWS[b], SG[b]).wait()

            @pl.when(s > 0)
            def _():
                pltpu.make_async_copy(SCB[b], acc.at[HV[b]], SS[b]).wait()

            for j in range(CE // 16):
                sl = pl.ds(j * 16, 16)
                HV[b][sl] = HN[b][sl]

            def grp(g, c):
                vals = AV[b][pl.ds(g * 16, 16)]
                for l in range(16):
                    e = g * 16 + l
                    val = vals[l]
                    for j in range(D // 16):
                        sl = pl.ds(j * 16, 16)
                        SCB[b][e, sl] = ROWS[b][e, sl] * val
                return c

            lax.fori_loop(0, CE // 16, grp, 0)
            pltpu.async_copy(SCB[b], acc.at[HV[b]], SS[b], add=True)

            @pl.when(k + 2 < nturn)
            def _():
                prep_idx(b, k + 2)

        def step(s, c):
            turn(s, 0, 2 * s, False)
            turn(s, 1, 2 * s + 1, nturn % 2 == 0 and False)
            return c

        lax.fori_loop(0, nturn // 2, step, 0)
        if nturn % 2:
            turn(jnp.int32(nturn // 2), 0, jnp.int32(nturn - 1), True)
        pltpu.make_async_copy(SCB[0], acc.at[HV[0]], SS[0]).wait()
        pltpu.make_async_copy(SCB[1], acc.at[HV[1]], SS[1]).wait()

        plsc.subcore_barrier()
        _unit_loop(sid, lambda off: pltpu.sync_copy(
            acc.at[pl.ds(off, U)], out_hbm.at[cid, pl.ds(off, U)]))

    return body


def _wspmm_scratch():
    return [
        pltpu.VMEM_SHARED((N, D), jnp.float32),
        pltpu.VMEM((CE,), jnp.int32),
        pltpu.VMEM((CE,), jnp.int32),
        pltpu.VMEM((CE,), jnp.int32),
        pltpu.VMEM((CE,), jnp.int32),
        pltpu.VMEM((CE,), jnp.int32),
        pltpu.VMEM((CE,), jnp.int32),
        pltpu.VMEM((CE,), jnp.float32),
        pltpu.VMEM((CE,), jnp.float32),
        pltpu.VMEM((CE, D), jnp.float32),
        pltpu.VMEM((CE, D), jnp.float32),
        pltpu.VMEM((CE, D), jnp.float32),
        pltpu.VMEM((CE, D), jnp.float32),
        pltpu.SemaphoreType.DMA,
        pltpu.SemaphoreType.DMA,
        pltpu.SemaphoreType.DMA,
        pltpu.SemaphoreType.DMA,
        pltpu.SemaphoreType.DMA,
        pltpu.SemaphoreType.DMA,
    ]


@jax.jit
def _spmm_sc(x, h_idx, t_idx, vals):
    k = pl.kernel(
        _make_wspmm_body(all_edges=False),
        out_type=jax.ShapeDtypeStruct((NC, N, D), jnp.float32),
        mesh=_MESH,
        compiler_params=_SC_PARAMS,
        scratch_types=_wspmm_scratch(),
    )
    return k(x, h_idx, t_idx, vals)


@jax.jit
def _apply_sc(x, h_idx, t_idx, alphas):
    k = pl.kernel(
        _make_wspmm_body(all_edges=True),
        out_type=jax.ShapeDtypeStruct((NC, N, D), jnp.float32),
        mesh=_MESH,
        compiler_params=_SC_PARAMS,
        scratch_types=_wspmm_scratch(),
    )
    return k(x, h_idx, t_idx, alphas)


def _alpha_body(cat_hbm, h_hbm, t_hbm, alpha_hbm, rs_hbm,
                stage, rsloc, hr0, hr1, th0, th1, tt0, tt1, av0, av1,
                rh0, rh1, rt0, rt1, pt, redbuf, outv,
                sg0, sg1, sw0, sw1, si0, si1):
    cid = lax.axis_index("c")
    sid = lax.axis_index("s")
    coff = cid * N
    HR = (hr0, hr1)
    TH = (th0, th1)
    TT = (tt0, tt1)
    AV = (av0, av1)
    RH = (rh0, rh1)
    RT = (rt0, rt1)
    SG = (sg0, sg1)
    SW = (sw0, sw1)
    SI = (si0, si1)

    def z(i, c):
        rsloc[pl.ds(i * 16, 16)] = jnp.zeros((16,), jnp.float32)
        return c

    lax.fori_loop(0, NPAD // 16, z, 0)

    def base_of(k):
        return (sid + k * NS) * CE

    def prep_idx(b, k):
        base = base_of(k)
        pltpu.async_copy(h_hbm.at[pl.ds(base, CE)], HR[b], SI[b])
        pltpu.async_copy(t_hbm.at[pl.ds(base, CE)], TT[b], SI[b])

    def launch(b, k):
        base = base_of(k)
        pltpu.make_async_copy(h_hbm.at[pl.ds(base, CE)], HR[b], SI[b]).wait()
        pltpu.make_async_copy(t_hbm.at[pl.ds(base, CE)], TT[b], SI[b]).wait()
        for j in range(CE // 16):
            sl = pl.ds(j * 16, 16)
            TH[b][sl] = HR[b][sl] + coff
            TT[b][sl] = TT[b][sl] + coff
        pltpu.async_copy(cat_hbm.at[TH[b]], RH[b], SG[b])
        pltpu.async_copy(cat_hbm.at[TT[b]], RT[b], SG[b])

    prep_idx(0, 0)
    prep_idx(1, 1)
    launch(0, jnp.int32(0))

    lanes = lax.broadcasted_iota(jnp.int32, (16,), 0)

    def turn(s, b):
        k = 2 * s + b
        bn = 1 - b
        base = base_of(k)

        @pl.when(k + 1 < NCHA)
        def _():
            launch(bn, k + 1)

        pltpu.make_async_copy(cat_hbm.at[TH[b]], RH[b], SG[b]).wait()
        pltpu.make_async_copy(cat_hbm.at[TT[b]], RT[b], SG[b]).wait()

        @pl.when(s > 0)
        def _():
            pltpu.make_async_copy(
                AV[b], alpha_hbm.at[pl.ds(cid * E + base, CE)], SW[b]).wait()

        def grp(g, c):
            for l in range(16):
                e = g * 16 + l
                p = RH[b][e, pl.ds(0, 16)] * RT[b][e, pl.ds(0, 16)]
                for j in range(1, D // 16):
                    sl = pl.ds(j * 16, 16)
                    p = p + RH[b][e, sl] * RT[b][e, sl]
                pt[l, :] = p
            sv = plsc.load_gather(pt, [lanes, jnp.zeros((16,), jnp.int32)])
            for j in range(1, 16):
                sv = sv + plsc.load_gather(pt, [lanes, jnp.full((16,), j, jnp.int32)])
            avec = (sv + 1.0) * 0.5
            AV[b][pl.ds(g * 16, 16)] = avec
            hvec = HR[b][pl.ds(g * 16, 16)]
            plsc.addupdate_scatter(rsloc, [hvec], avec)
            return c

        lax.fori_loop(0, CE // 16, grp, 0)
        pltpu.async_copy(AV[b], alpha_hbm.at[pl.ds(cid * E + base, CE)], SW[b])

        @pl.when(k + 2 < NCHA)
        def _():
            prep_idx(b, k + 2)

    def step(s, c):
        turn(s, 0)
        turn(s, 1)
        return c

    lax.fori_loop(0, NCHA // 2, step, 0)
    for b in range(2):
        base = base_of(jnp.int32(NCHA - 2 + b))
        pltpu.make_async_copy(
            AV[b], alpha_hbm.at[pl.ds(cid * E + base, CE)], SW[b]).wait()

    pltpu.sync_copy(rsloc, stage.at[sid])
    plsc.subcore_barrier()
    pltpu.sync_copy(stage.at[:, pl.ds(sid * 640, 640)], redbuf)

    def red(kk, c):
        sl = pl.ds(kk * 16, 16)
        sv = redbuf[0, sl]
        for r in range(1, 16):
            sv = sv + redbuf[r, sl]
        outv[sl] = sv
        return c

    lax.fori_loop(0, 40, red, 0)
    pltpu.sync_copy(outv, rs_hbm.at[pl.ds(cid * NPAD + sid * 640, 640)])


@jax.jit
def _alpha_sc(cat, h_idx, t_idx):
    k = pl.kernel(
        _alpha_body,
        out_type=(jax.ShapeDtypeStruct((NC * E,), jnp.float32),
                  jax.ShapeDtypeStruct((NC * NPAD,), jnp.float32)),
        mesh=_MESH,
        compiler_params=_SC_PARAMS,
        scratch_types=[
            pltpu.VMEM_SHARED((NS, NPAD), jnp.float32),
            pltpu.VMEM((NPAD,), jnp.float32),
            pltpu.VMEM((CE,), jnp.int32),
            pltpu.VMEM((CE,), jnp.int32),
            pltpu.VMEM((CE,), jnp.int32),
            pltpu.VMEM((CE,), jnp.int32),
            pltpu.VMEM((CE,), jnp.int32),
            pltpu.VMEM((CE,), jnp.int32),
            pltpu.VMEM((CE,), jnp.float32),
            pltpu.VMEM((CE,), jnp.float32),
            pltpu.VMEM((CE, D), jnp.float32),
            pltpu.VMEM((CE, D), jnp.float32),
            pltpu.VMEM((CE, D), jnp.float32),
            pltpu.VMEM((CE, D), jnp.float32),
            pltpu.VMEM((16, 16), jnp.float32),
            pltpu.VMEM((16, 640), jnp.float32),
            pltpu.VMEM((640,), jnp.float32),
            pltpu.SemaphoreType.DMA,
            pltpu.SemaphoreType.DMA,
            pltpu.SemaphoreType.DMA,
            pltpu.SemaphoreType.DMA,
            pltpu.SemaphoreType.DMA,
            pltpu.SemaphoreType.DMA,
        ],
    )
    return k(cat, h_idx, t_idx)


RB = 1000


def _dense_body(x_ref, p0_ref, p1_ref, uw_ref, iw_ref,
                gnn_ref, int_ref, gn_ref, in_ref):
    pid = pl.program_id(0)
    g = p0_ref[...] + p1_ref[...]
    w = jnp.where(pid < (NU // RB), uw_ref[...], iw_ref[...])
    logits = jnp.dot(x_ref[...], w, preferred_element_type=jnp.float32)
    m = jnp.max(logits, axis=1, keepdims=True)
    ee = jnp.exp(logits - m)
    sm = ee / jnp.sum(ee, axis=1, keepdims=True)
    il = lax.dot_general(sm, w, dimension_numbers=(((1,), (1,)), ((), ())),
                         preferred_element_type=jnp.float32)
    gnn_ref[...] = g
    int_ref[...] = il
    gn_ref[...] = g / jnp.maximum(
        jnp.sqrt(jnp.sum(g * g, axis=1, keepdims=True)), 1e-12)
    in_ref[...] = il / jnp.maximum(
        jnp.sqrt(jnp.sum(il * il, axis=1, keepdims=True)), 1e-12)


@jax.jit
def _dense_tc(x, gnn_p, user_intent, item_intent):
    blk = pl.BlockSpec((RB, D), lambda i: (i, 0))
    wblk = pl.BlockSpec((D, D), lambda i: (0, 0))
    sds = jax.ShapeDtypeStruct((N, D), jnp.float32)
    return pl.pallas_call(
        _dense_body,
        grid=(N // RB,),
        in_specs=[blk, blk, blk, wblk, wblk],
        out_specs=[blk, blk, blk, blk],
        out_shape=[sds, sds, sds, sds],
    )(x, gnn_p[0], gnn_p[1], user_intent, item_intent)


def _combine_body(gnn_ref, int_ref, ag_ref, ai_ref, rsg_ref, rsi_ref, x_ref,
                  gaa_ref, iaa_ref, xn_ref):
    rsg = rsg_ref[...]
    rsi = rsi_ref[...]
    dg = jnp.where(rsg > 0, 1.0 / rsg, 0.0)
    di = jnp.where(rsi > 0, 1.0 / rsi, 0.0)
    gaa = dg * ag_ref[...]
    iaa = di * ai_ref[...]
    gaa_ref[...] = gaa
    iaa_ref[...] = iaa
    xn_ref[...] = gnn_ref[...] + int_ref[...] + gaa + iaa + x_ref[...]


def _combine_total_body(gnn_ref, int_ref, ag_ref, ai_ref, rsg_ref, rsi_ref,
                        x_ref, x0_ref, gaa_ref, iaa_ref, xn_ref, tot_ref):
    _combine_body(gnn_ref, int_ref, ag_ref, ai_ref, rsg_ref, rsi_ref, x_ref,
                  gaa_ref, iaa_ref, xn_ref)
    tot_ref[...] = x0_ref[...] + x_ref[...] + xn_ref[...]


@functools.partial(jax.jit, static_argnames=("with_total",))
def _combine_tc(gnn, intl, accp, rsg, rsi, x, x0=None, with_total=False):
    blk = pl.BlockSpec((RB, D), lambda i: (i, 0))
    rblk = pl.BlockSpec((RB, 1), lambda i: (i, 0))
    sds = jax.ShapeDtypeStruct((N, D), jnp.float32)
    nout = 3 + (1 if with_total else 0)
    body = _combine_total_body if with_total else _combine_body
    args = [gnn, intl, accp[0], accp[1], rsg, rsi, x]
    if with_total:
        args.append(x0)
    nblk = len(args) - 6
    return pl.pallas_call(
        body,
        grid=(N // RB,),
        in_specs=[blk] * 4 + [rblk] * 2 + [blk] * nblk,
        out_specs=[blk] * nout,
        out_shape=[sds] * nout,
    )(*args)


def kernel(user_emb, item_emb, user_intent, item_intent, norm_vals,
           h_idx, t_idx, users, items):
    x0 = jnp.concatenate([user_emb, item_emb], axis=0)
    h_idx = h_idx.astype(jnp.int32)
    t_idx = t_idx.astype(jnp.int32)

    x = x0
    gnn_l, int_l, gaa_l, iaa_l = [], [], [], []
    for layer in range(2):
        gnn_p = _spmm_sc(x, h_idx, t_idx, norm_vals)
        gnn, intl, gn, inn = _dense_tc(x, gnn_p, user_intent, item_intent)
        cat = jnp.concatenate([gn, inn], axis=0)
        alphas, rs = _alpha_sc(cat, h_idx, t_idx)
        accp = _apply_sc(x, h_idx, t_idx, alphas)
        rsg = rs[:N][:, None]
        rsi = rs[NPAD:NPAD + N][:, None]
        if layer == 0:
            gaa, iaa, xn = _combine_tc(gnn, intl, accp, rsg, rsi, x)
        else:
            gaa, iaa, xn, tot = _combine_tc(gnn, intl, accp, rsg, rsi, x, x0,
                                            with_total=True)
        gnn_l.append(gnn)
        int_l.append(intl)
        gaa_l.append(gaa)
        iaa_l.append(iaa)
        x = xn

    return (tot[:NU], tot[NU:], tuple(gnn_l), tuple(int_l),
            tuple(gaa_l), tuple(iaa_l))
